# baseline (speedup 1.0000x reference)
import jax
import jax.numpy as jnp
from jax.experimental import pallas as pl
from jax.experimental.pallas import tpu as pltpu

_LANE = 128
_SUBLANE = 8
_TB = 4096


def _round_up(x, m):
    return ((x + m - 1) // m) * m


def _pad2d(a, rows, cols):
    pr, pc = rows - a.shape[0], cols - a.shape[1]
    if pr == 0 and pc == 0:
        return a
    return jnp.pad(a, ((0, pr), (0, pc)))


def _mlp_kernel(x_ref, w0_h, b0_h, w1_h, b1_h, w2_h, b2_h, o_ref,
                w0_v, b0_v, w1_v, b1_v, w2_v, b2_v, sems):
    pairs = ((w0_h, w0_v), (b0_h, b0_v), (w1_h, w1_v),
             (b1_h, b1_v), (w2_h, w2_v), (b2_h, b2_v))

    @pl.when(pl.program_id(0) == 0)
    def _load_params():
        for k, (src, dst) in enumerate(pairs):
            pltpu.make_async_copy(src, dst, sems.at[k]).start()
        for k, (src, dst) in enumerate(pairs):
            pltpu.make_async_copy(src, dst, sems.at[k]).wait()

    h = jnp.dot(x_ref[...], w0_v[...], preferred_element_type=jnp.float32)
    h = jnp.tanh(h + b0_v[...])
    h = jnp.dot(h, w1_v[...], preferred_element_type=jnp.float32)
    h = jnp.tanh(h + b1_v[...])
    y = jnp.dot(h, w2_v[...], preferred_element_type=jnp.float32)
    o_ref[...] = y + b2_v[...]


def kernel(x, w0, b0, w1, b1, w2, b2):
    B, D0 = x.shape
    dims = [D0, w0.shape[1], w1.shape[1], w2.shape[1]]
    dp = [_round_up(d, _LANE) for d in dims]

    tb = min(_round_up(B, _SUBLANE), _TB)
    B_pad = _round_up(B, tb)

    x_p = _pad2d(x, B_pad, dp[0])
    ws = [_pad2d(w, dp[k], dp[k + 1]) for k, w in enumerate((w0, w1, w2))]
    bs = [_pad2d(b.reshape(1, -1), 1, dp[k + 1])
          for k, b in enumerate((b0, b1, b2))]

    any_spec = pl.BlockSpec(memory_space=pl.ANY)
    in_specs = [pl.BlockSpec((tb, dp[0]), lambda i: (i, 0))]
    in_specs += [any_spec] * 6

    scratch_shapes = [
        pltpu.VMEM((dp[0], dp[1]), jnp.float32),
        pltpu.VMEM((1, dp[1]), jnp.float32),
        pltpu.VMEM((dp[1], dp[2]), jnp.float32),
        pltpu.VMEM((1, dp[2]), jnp.float32),
        pltpu.VMEM((dp[2], dp[3]), jnp.float32),
        pltpu.VMEM((1, dp[3]), jnp.float32),
        pltpu.SemaphoreType.DMA((6,)),
    ]

    out = pl.pallas_call(
        _mlp_kernel,
        out_shape=jax.ShapeDtypeStruct((B_pad, dp[3]), x.dtype),
        grid=(B_pad // tb,),
        in_specs=in_specs,
        out_specs=pl.BlockSpec((tb, dp[3]), lambda i: (i, 0)),
        scratch_shapes=scratch_shapes,
        compiler_params=pltpu.CompilerParams(
            dimension_semantics=("arbitrary",),
            vmem_limit_bytes=64 * 1024 * 1024),
    )(x_p, ws[0], bs[0], ws[1], bs[1], ws[2], bs[2])
    return out[:B, :dims[3]]


# speedup vs baseline: 1.0612x; 1.0612x over previous
import jax
import jax.numpy as jnp
from jax.experimental import pallas as pl
from jax.experimental.pallas import tpu as pltpu

_LANE = 128
_SUBLANE = 8
_TB = 4096


def _round_up(x, m):
    return ((x + m - 1) // m) * m


def _pad2d(a, rows, cols):
    pr, pc = rows - a.shape[0], cols - a.shape[1]
    if pr == 0 and pc == 0:
        return a
    return jnp.pad(a, ((0, pr), (0, pc)))


def _mlp_kernel(x_ref, w0_ref, b0_ref, w1_ref, b1_ref, w2_ref, b2_ref, o_ref):
    x = x_ref[...].astype(jnp.bfloat16)
    h = jnp.dot(x, w0_ref[...].astype(jnp.bfloat16),
                preferred_element_type=jnp.float32)
    h = jnp.tanh(h + b0_ref[...]).astype(jnp.bfloat16)
    h = jnp.dot(h, w1_ref[...].astype(jnp.bfloat16),
                preferred_element_type=jnp.float32)
    h = jnp.tanh(h + b1_ref[...]).astype(jnp.bfloat16)
    y = jnp.dot(h, w2_ref[...].astype(jnp.bfloat16),
                preferred_element_type=jnp.float32)
    o_ref[...] = y + b2_ref[...]


def kernel(x, w0, b0, w1, b1, w2, b2):
    B, D0 = x.shape
    dims = [D0, w0.shape[1], w1.shape[1], w2.shape[1]]
    dp = [_round_up(d, _LANE) for d in dims]

    tb = min(_round_up(B, _SUBLANE), _TB)
    B_pad = _round_up(B, tb)

    x_p = _pad2d(x, B_pad, dp[0])
    ws = []
    for k, w in enumerate((w0, w1, w2)):
        ws.append(_pad2d(w, dp[k], dp[k + 1]))
    bs = [
        _pad2d(b.reshape(1, -1), 1, dp[k + 1])
        for k, b in enumerate((b0, b1, b2))
    ]

    in_specs = [pl.BlockSpec((tb, dp[0]), lambda i: (i, 0))]
    for k in range(3):
        in_specs.append(pl.BlockSpec((dp[k], dp[k + 1]), lambda i: (0, 0)))
        in_specs.append(pl.BlockSpec((1, dp[k + 1]), lambda i: (0, 0)))

    out = pl.pallas_call(
        _mlp_kernel,
        out_shape=jax.ShapeDtypeStruct((B_pad, dp[3]), x.dtype),
        grid=(B_pad // tb,),
        in_specs=in_specs,
        out_specs=pl.BlockSpec((tb, dp[3]), lambda i: (i, 0)),
        compiler_params=pltpu.CompilerParams(
            dimension_semantics=("parallel",),
            vmem_limit_bytes=64 * 1024 * 1024),
    )(x_p, ws[0], bs[0], ws[1], bs[1], ws[2], bs[2])
    return out[:B, :dims[3]]


# speedup vs baseline: 1.0730x; 1.0112x over previous
import jax
import jax.numpy as jnp
from jax.experimental import pallas as pl
from jax.experimental.pallas import tpu as pltpu

_LANE = 128
_SUBLANE = 8
_TB = 4096


def _round_up(x, m):
    return ((x + m - 1) // m) * m


def _pad2d(a, rows, cols):
    pr, pc = rows - a.shape[0], cols - a.shape[1]
    if pr == 0 and pc == 0:
        return a
    return jnp.pad(a, ((0, pr), (0, pc)))


def _mlp_kernel(x_ref, w0_ref, b0_ref, w1_ref, b1_ref, w2_ref, b2_ref, o_ref):
    h = jnp.dot(x_ref[...], w0_ref[...], preferred_element_type=jnp.float32)
    h = jnp.tanh(h + b0_ref[...])
    h = jnp.dot(h, w1_ref[...], preferred_element_type=jnp.float32)
    h = jnp.tanh(h + b1_ref[...])
    y = jnp.dot(h, w2_ref[...], preferred_element_type=jnp.float32)
    o_ref[...] = y + b2_ref[...]


def kernel(x, w0, b0, w1, b1, w2, b2):
    B, D0 = x.shape
    dims = [D0, w0.shape[1], w1.shape[1], w2.shape[1]]
    dp = [_round_up(d, _LANE) for d in dims]

    tb = min(_round_up(B, _SUBLANE), _TB)
    B_pad = _round_up(B, tb)

    x_p = _pad2d(x, B_pad, dp[0])
    ws = []
    for k, w in enumerate((w0, w1, w2)):
        ws.append(_pad2d(w, dp[k], dp[k + 1]))
    bs = [
        _pad2d(b.reshape(1, -1), 1, dp[k + 1])
        for k, b in enumerate((b0, b1, b2))
    ]

    in_specs = [pl.BlockSpec((tb, dp[0]), lambda i: (i, 0))]
    for k in range(3):
        in_specs.append(pl.BlockSpec((dp[k], dp[k + 1]), lambda i: (0, 0)))
        in_specs.append(pl.BlockSpec((1, dp[k + 1]), lambda i: (0, 0)))

    out = pl.pallas_call(
        _mlp_kernel,
        out_shape=jax.ShapeDtypeStruct((B_pad, dp[3]), x.dtype),
        grid=(B_pad // tb,),
        in_specs=in_specs,
        out_specs=pl.BlockSpec((tb, dp[3]), lambda i: (i, 0)),
        compiler_params=pltpu.CompilerParams(
            dimension_semantics=("parallel",),
            vmem_limit_bytes=64 * 1024 * 1024),
    )(x_p, ws[0], bs[0], ws[1], bs[1], ws[2], bs[2])
    return out[:B, :dims[3]]
